# baseline (speedup 1.0000x reference)
import functools

import jax
import jax.numpy as jnp
from jax import lax
from jax.experimental import pallas as pl
from jax.experimental.pallas import tpu as pltpu
from jax.experimental.pallas import tpu_sc as plsc

_L = 16
_BT = 512
_BT_M = 512
_BR = 16


def _tc_body(x_ref, emb_ref, o_ref):
    o_ref[...] = x_ref[...] + emb_ref[...][None]


def _sc_body(x_hbm, emb_hbm, o_hbm, *, nblk, batch, srow_blk, trow_blk, d):
    grp = 8

    def block_body(x_v, emb_v, o_v):
        for r in range(_BR):
            for g in range(0, d, grp * _L):
                xs = [x_v.at[r, pl.ds(g + i * _L, _L)][...] for i in range(grp)]
                es = [emb_v.at[r, pl.ds(g + i * _L, _L)][...] for i in range(grp)]
                for i in range(grp):
                    o_v.at[r, pl.ds(g + i * _L, _L)][...] = xs[i] + es[i]

    pltpu.emit_pipeline(
        block_body,
        grid=(nblk, batch),
        in_specs=[
            pl.BlockSpec((_BR, d), index_map=lambda i, b: (b * trow_blk + srow_blk + i, 0)),
            pl.BlockSpec((_BR, d), index_map=lambda i, b: (srow_blk + i, 0)),
        ],
        out_specs=[pl.BlockSpec((_BR, d), index_map=lambda i, b: (b * nblk + i, 0))],
        core_axis_name=("c", "s"),
        dimension_semantics=(pltpu.PARALLEL, pltpu.ARBITRARY),
    )(x_hbm, emb_hbm, o_hbm)


def kernel(x, emb):
    B, T, D = x.shape
    S = 6656

    tc_out = pl.pallas_call(
        _tc_body,
        grid=(S // _BT,),
        in_specs=[
            pl.BlockSpec((B, _BT, D), lambda i: (0, i, 0)),
            pl.BlockSpec((_BT, D), lambda i: (i, 0)),
        ],
        out_specs=pl.BlockSpec((B, _BT, D), lambda i: (0, i, 0)),
        out_shape=jax.ShapeDtypeStruct((B, T, D), x.dtype),
    )(x, emb)

    q_rows = T - S
    x2 = x.reshape(B * T, D)
    mesh = plsc.VectorSubcoreMesh(core_axis_name="c", subcore_axis_name="s")
    sc_body = functools.partial(
        _sc_body, nblk=q_rows // _BR, batch=B,
        srow_blk=S // _BR, trow_blk=T // _BR, d=D)
    sc_out = pl.kernel(
        sc_body,
        out_type=jax.ShapeDtypeStruct((B * q_rows, D), x.dtype),
        mesh=mesh,
    )(x2, emb)

    def _merge_body(full_ref, sc_ref, o_ref):
        o_ref[...] = sc_ref[...]

    return pl.pallas_call(
        _merge_body,
        grid=(q_rows // _BT_M,),
        in_specs=[
            pl.BlockSpec((1, 8, 128), lambda i: (0, 0, 0)),
            pl.BlockSpec((B, _BT_M, D), lambda i: (0, i, 0)),
        ],
        out_specs=pl.BlockSpec(
            (B, _BT_M, D), lambda i: (0, (S // _BT_M) + i, 0)),
        out_shape=jax.ShapeDtypeStruct((B, T, D), x.dtype),
        input_output_aliases={0: 0},
    )(tc_out, sc_out.reshape(B, q_rows, D))


# speedup vs baseline: 1.0775x; 1.0775x over previous
import functools

import jax
import jax.numpy as jnp
from jax import lax
from jax.experimental import pallas as pl
from jax.experimental.pallas import tpu as pltpu
from jax.experimental.pallas import tpu_sc as plsc

_L = 16
_BT = 512
_BT_M = 512
_BR = 16


def _tc_body(x_ref, emb_ref, o_ref):
    o_ref[...] = x_ref[...] + emb_ref[...][None]


def _sc_body(x_hbm, emb_hbm, o_hbm, *, nblk, batch, srow_blk, trow_blk, d):
    grp = 8

    def block_body(x_v, emb_v, o_v):
        for r in range(_BR):
            for g in range(0, d, grp * _L):
                xs = [x_v.at[r, pl.ds(g + i * _L, _L)][...] for i in range(grp)]
                es = [emb_v.at[r, pl.ds(g + i * _L, _L)][...] for i in range(grp)]
                for i in range(grp):
                    o_v.at[r, pl.ds(g + i * _L, _L)][...] = xs[i] + es[i]

    pltpu.emit_pipeline(
        block_body,
        grid=(nblk, batch),
        in_specs=[
            pl.BlockSpec((_BR, d), index_map=lambda i, b: (b * trow_blk + srow_blk + i, 0)),
            pl.BlockSpec((_BR, d), index_map=lambda i, b: (srow_blk + i, 0)),
        ],
        out_specs=[pl.BlockSpec((_BR, d), index_map=lambda i, b: (b * nblk + i, 0))],
        core_axis_name=("c", "s"),
        dimension_semantics=(pltpu.PARALLEL, pltpu.ARBITRARY),
    )(x_hbm, emb_hbm, o_hbm)


def kernel(x, emb):
    B, T, D = x.shape
    S = 7680

    tc_out = pl.pallas_call(
        _tc_body,
        grid=(S // _BT,),
        in_specs=[
            pl.BlockSpec((B, _BT, D), lambda i: (0, i, 0)),
            pl.BlockSpec((_BT, D), lambda i: (i, 0)),
        ],
        out_specs=pl.BlockSpec((B, _BT, D), lambda i: (0, i, 0)),
        out_shape=jax.ShapeDtypeStruct((B, T, D), x.dtype),
    )(x, emb)

    q_rows = T - S
    x2 = x.reshape(B * T, D)
    mesh = plsc.VectorSubcoreMesh(core_axis_name="c", subcore_axis_name="s")
    sc_body = functools.partial(
        _sc_body, nblk=q_rows // _BR, batch=B,
        srow_blk=S // _BR, trow_blk=T // _BR, d=D)
    sc_out = pl.kernel(
        sc_body,
        out_type=jax.ShapeDtypeStruct((B * q_rows, D), x.dtype),
        mesh=mesh,
    )(x2, emb)

    def _merge_body(full_ref, sc_ref, o_ref):
        o_ref[...] = sc_ref[...]

    return pl.pallas_call(
        _merge_body,
        grid=(q_rows // _BT_M,),
        in_specs=[
            pl.BlockSpec((1, 8, 128), lambda i: (0, 0, 0)),
            pl.BlockSpec((B, _BT_M, D), lambda i: (0, i, 0)),
        ],
        out_specs=pl.BlockSpec(
            (B, _BT_M, D), lambda i: (0, (S // _BT_M) + i, 0)),
        out_shape=jax.ShapeDtypeStruct((B, T, D), x.dtype),
        input_output_aliases={0: 0},
    )(tc_out, sc_out.reshape(B, q_rows, D))


# speedup vs baseline: 1.3483x; 1.2514x over previous
import functools

import jax
import jax.numpy as jnp
from jax import lax
from jax.experimental import pallas as pl
from jax.experimental.pallas import tpu as pltpu
from jax.experimental.pallas import tpu_sc as plsc

_L = 16
_BT = 512
_BT_M = 512
_BR = 16


def _tc_body(x_ref, emb_ref, o_ref):
    o_ref[...] = x_ref[...] + emb_ref[...][None]


def _sc_body(x_hbm, emb_hbm, o_hbm, *, nblk, batch, srow_blk, trow_blk, d):
    grp = 8

    def block_body(x_v, emb_v, o_v):
        for r in range(_BR):
            for g in range(0, d, grp * _L):
                xs = [x_v.at[r, pl.ds(g + i * _L, _L)][...] for i in range(grp)]
                es = [emb_v.at[r, pl.ds(g + i * _L, _L)][...] for i in range(grp)]
                for i in range(grp):
                    o_v.at[r, pl.ds(g + i * _L, _L)][...] = xs[i] + es[i]

    pltpu.emit_pipeline(
        block_body,
        grid=(nblk, batch),
        in_specs=[
            pl.BlockSpec((_BR, d), index_map=lambda i, b: (b * trow_blk + srow_blk + i, 0)),
            pl.BlockSpec((_BR, d), index_map=lambda i, b: (srow_blk + i, 0)),
        ],
        out_specs=[pl.BlockSpec((_BR, d), index_map=lambda i, b: (b * nblk + i, 0))],
        core_axis_name=("c", "s"),
        dimension_semantics=(pltpu.PARALLEL, pltpu.ARBITRARY),
    )(x_hbm, emb_hbm, o_hbm)


def kernel(x, emb):
    B, T, D = x.shape
    S = 8192

    tc_out = pl.pallas_call(
        _tc_body,
        grid=(S // _BT,),
        in_specs=[
            pl.BlockSpec((B, _BT, D), lambda i: (0, i, 0)),
            pl.BlockSpec((_BT, D), lambda i: (i, 0)),
        ],
        out_specs=pl.BlockSpec((B, _BT, D), lambda i: (0, i, 0)),
        out_shape=jax.ShapeDtypeStruct((B, T, D), x.dtype),
    )(x, emb)

    return tc_out
    q_rows = T - S
    x2 = x.reshape(B * T, D)
    mesh = plsc.VectorSubcoreMesh(core_axis_name="c", subcore_axis_name="s")
    sc_body = functools.partial(
        _sc_body, nblk=q_rows // _BR, batch=B,
        srow_blk=S // _BR, trow_blk=T // _BR, d=D)
    sc_out = pl.kernel(
        sc_body,
        out_type=jax.ShapeDtypeStruct((B * q_rows, D), x.dtype),
        mesh=mesh,
    )(x2, emb)

    def _merge_body(full_ref, sc_ref, o_ref):
        o_ref[...] = sc_ref[...]

    return pl.pallas_call(
        _merge_body,
        grid=(q_rows // _BT_M,),
        in_specs=[
            pl.BlockSpec((1, 8, 128), lambda i: (0, 0, 0)),
            pl.BlockSpec((B, _BT_M, D), lambda i: (0, i, 0)),
        ],
        out_specs=pl.BlockSpec(
            (B, _BT_M, D), lambda i: (0, (S // _BT_M) + i, 0)),
        out_shape=jax.ShapeDtypeStruct((B, T, D), x.dtype),
        input_output_aliases={0: 0},
    )(tc_out, sc_out.reshape(B, q_rows, D))
